# baseline (speedup 1.0000x reference)
import functools

import jax
import jax.numpy as jnp
from jax.experimental import pallas as pl
from jax.experimental.pallas import tpu as pltpu

_N = 32
_A = 8732
_C = 81


def _fused_body(cls_ref, ct_ref, lp_ref, lt_ref, out_ref, acc):
    step = pl.program_id(0)

    @pl.when(step == 0)
    def _init():
        acc[0] = 0.0
        acc[1] = 0.0
        acc[2] = 0.0

    rio8 = jax.lax.broadcasted_iota(jnp.int32, (8, _A), 0)
    ct8 = ct_ref[...]
    t_lane = jnp.sum(jnp.where(rio8 == step % 8, ct8, 0),
                     axis=0, keepdims=True)
    t_col = t_lane.reshape(_A, 1)

    x = cls_ref[0]
    m = jnp.max(x, axis=1, keepdims=True)
    e = jnp.exp(x - m)
    s = jnp.sum(e, axis=1, keepdims=True)
    cio = jax.lax.broadcasted_iota(jnp.int32, (_A, _C), 1)
    tl = jnp.sum(jnp.where(cio == t_col, x, 0.0), axis=1, keepdims=True)
    cl_col = (m - tl) + jnp.log(s)
    cl = cl_col.reshape(1, _A)

    pos = t_lane > 0
    posf = pos.astype(jnp.float32)
    np_row = jnp.sum(posf)
    k = jnp.minimum(3 * np_row.astype(jnp.int32), _A)
    pcl = jnp.sum(cl * posf)
    v = jnp.where(pos, 0.0, cl)
    sumv = jnp.sum(v)

    def bit_step(i, p):
        cand = p | (1 << (30 - i))
        tval = jax.lax.bitcast_convert_type(cand, jnp.float32)
        cnt = jnp.sum((v >= tval).astype(jnp.int32))
        return jnp.where(cnt >= k, cand, p)

    def searched_top(_):
        p = jax.lax.fori_loop(0, 31, bit_step, jnp.int32(0))
        tval = jax.lax.bitcast_convert_type(p, jnp.float32)
        gt = v > tval
        c = jnp.sum(gt.astype(jnp.int32))
        top = jnp.sum(jnp.where(gt, v, 0.0)) + tval * (k - c).astype(jnp.float32)
        return jnp.where(k == 0, 0.0, top)

    top = jax.lax.cond(k >= _A, lambda _: sumv, searched_top, 0)

    coff = 4 * (step % 2)
    d = lp_ref[...] - lt_ref[...]
    ad = jnp.abs(d)
    sl1 = jnp.where(ad < 1.0, 0.5 * d * d, ad - 0.5)
    rsel = (rio8 >= coff) & (rio8 < coff + 4)
    sl1 = jnp.where(rsel, sl1, 0.0)
    loc_row = jnp.sum(jnp.sum(sl1, axis=0, keepdims=True) * posf)

    acc[0] = acc[0] + pcl + top
    acc[1] = acc[1] + loc_row
    acc[2] = acc[2] + np_row

    @pl.when(step == _N - 1)
    def _fin():
        out_ref[...] = ((acc[0] + acc[1]) / acc[2]).reshape(1, 1)


@functools.partial(jax.jit)
def kernel(loc_preds, loc_targets, cls_preds, cls_targets):
    lp_t = loc_preds.transpose(0, 2, 1).reshape(4 * _N, _A)
    lt_t = loc_targets.transpose(0, 2, 1).reshape(4 * _N, _A)
    out = pl.pallas_call(
        _fused_body,
        grid=(_N,),
        in_specs=[
            pl.BlockSpec((1, _A, _C), lambda n: (n, 0, 0)),
            pl.BlockSpec((8, _A), lambda n: (n // 8, 0)),
            pl.BlockSpec((8, _A), lambda n: (n // 2, 0)),
            pl.BlockSpec((8, _A), lambda n: (n // 2, 0)),
        ],
        out_specs=pl.BlockSpec((1, 1), lambda n: (0, 0)),
        out_shape=jax.ShapeDtypeStruct((1, 1), jnp.float32),
        scratch_shapes=[pltpu.SMEM((4,), jnp.float32)],
    )(cls_preds, cls_targets, lp_t, lt_t)
    return out[0, 0]
